# baseline (speedup 1.0000x reference)
import functools

import jax
import jax.numpy as jnp
from jax import lax
from jax.experimental import pallas as pl
from jax.experimental.pallas import tpu as pltpu
from jax.experimental.pallas import tpu_sc as plsc

D_V = 1_000_000
D_M = 64
B = 16384

NC = 2
NS = 16
NW = NC * NS
ROWS_PER_W = D_M // NW
NROW = 8
NCOL = B // NROW


def _body(idx_hbm, table_hbm, out_hbm, idx_v, row_buf, sem):
    cid = lax.axis_index("c")
    sid = lax.axis_index("s")
    wid = sid * NC + cid

    pltpu.sync_copy(idx_hbm, idx_v)

    copies = []
    for rr in range(ROWS_PER_W):
        r = wid * ROWS_PER_W + rr
        row = table_hbm.at[r]
        for j in range(NROW):
            c = pltpu.make_async_copy(
                row.at[idx_v.at[j]], row_buf.at[rr].at[j], sem
            )
            c.start()
            copies.append(c)
    for c in copies:
        c.wait()
    for rr in range(ROWS_PER_W):
        r = wid * ROWS_PER_W + rr
        pltpu.sync_copy(row_buf.at[rr], out_hbm.at[r])


def kernel(token_indices, lookup):
    idx2d = token_indices.astype(jnp.int32).reshape(NROW, NCOL)
    flat = jnp.reshape(lookup, (D_M * D_V // 128, 128))
    flat = lax.optimization_barrier(flat)
    table_lin = jnp.reshape(flat, (D_M, D_V))
    mesh = plsc.VectorSubcoreMesh(core_axis_name="c", subcore_axis_name="s")
    k = functools.partial(
        pl.kernel,
        mesh=mesh,
        out_type=jax.ShapeDtypeStruct((D_M, NROW, NCOL), jnp.float32),
        scratch_types=[
            pltpu.VMEM((NROW, NCOL), jnp.int32),
            pltpu.VMEM((ROWS_PER_W, NROW, NCOL), jnp.float32),
            pltpu.SemaphoreType.DMA,
        ],
        compiler_params=pltpu.CompilerParams(use_tc_tiling_on_sc=False),
    )(_body)
    out3 = k(idx2d, table_lin)
    return out3.reshape(D_M, B)


# speedup vs baseline: 10.9383x; 10.9383x over previous
import functools

import jax
import jax.numpy as jnp
from jax import lax
from jax.experimental import pallas as pl
from jax.experimental.pallas import tpu as pltpu
from jax.experimental.pallas import tpu_sc as plsc

D_V = 1_000_000
D_M = 64
B = 16384

NC = 2
NS = 16
NW = NC * NS
SEG = B // NW


def _body(idx_hbm, tableT_hbm, out_hbm, idx_v, row_ids_v, rows_v, sem, gsem):
    cid = lax.axis_index("c")
    sid = lax.axis_index("s")
    wid = sid * NC + cid
    base = wid * SEG

    pltpu.sync_copy(idx_hbm.at[pl.ds(base, SEG)], idx_v)

    def to_rows(v, carry):
        row_ids_v[pl.ds(v * 16, 16)] = lax.shift_right_logical(
            idx_v[pl.ds(v * 16, 16)], 1
        )
        return carry

    lax.fori_loop(0, SEG // 16, to_rows, 0)

    src = tableT_hbm.at[row_ids_v]
    pltpu.make_async_copy(src, rows_v, gsem).start()
    pltpu.make_async_copy(src, rows_v, gsem).wait()

    pltpu.sync_copy(rows_v, out_hbm.at[pl.ds(base, SEG), :])


def kernel(token_indices, lookup):
    idx = token_indices.astype(jnp.int32)
    tableT = jnp.transpose(lookup).reshape(D_V // 2, 2 * D_M)
    mesh = plsc.VectorSubcoreMesh(core_axis_name="c", subcore_axis_name="s")
    k = functools.partial(
        pl.kernel,
        mesh=mesh,
        out_type=jax.ShapeDtypeStruct((B, 2 * D_M), jnp.float32),
        scratch_types=[
            pltpu.VMEM((SEG,), jnp.int32),
            pltpu.VMEM((SEG,), jnp.int32),
            pltpu.VMEM((SEG, 2 * D_M), jnp.float32),
            pltpu.SemaphoreType.DMA,
            pltpu.SemaphoreType.DMA,
        ],
    )(_body)
    pairs = k(idx, tableT)
    odd = (idx & 1)[:, None] == 1
    halves = jnp.where(odd, pairs[:, D_M:], pairs[:, :D_M])
    return halves.T
